# baseline (speedup 1.0000x reference)
import functools

import jax
import jax.numpy as jnp
from jax import lax
from jax.experimental import pallas as pl
from jax.experimental.pallas import tpu as pltpu
from jax.experimental.pallas import tpu_sc as plsc

N_NODES = 1000000
DIM = 16
BATCH = 16384
L = 16

NC = 2
NS = 16
NW = NC * NS

NTC = (N_NODES + 127) // 128
FLAT = NTC * DIM * 128
TCW = (NTC + NW - 1) // NW
NODES_W = TCW * 128
T_SIZE = NODES_W

CCOLS = 8
CNODES = CCOLS * 128
CELEMS = CCOLS * DIM * 128
NCH = (TCW + CCOLS - 1) // CCOLS
SB = 128


TCG = 96
C_BLK = TCG * 128
_TGRID = (N_NODES + C_BLK - 1) // C_BLK


def _to_tiles_body(x_ref, o_ref):
    x = x_ref[...]
    o_ref[...] = jnp.transpose(jnp.reshape(x, (DIM, TCG, 128)), (1, 0, 2))


_to_tiles = pl.pallas_call(
    _to_tiles_body,
    grid=(_TGRID,),
    in_specs=[pl.BlockSpec((DIM, C_BLK), lambda j: (0, j))],
    out_specs=pl.BlockSpec((TCG, DIM, 128), lambda j: (j, 0, 0)),
    out_shape=jax.ShapeDtypeStruct((NTC, DIM, 128), jnp.float32),
)


def _from_tiles_body(z_ref, o_ref):
    z = z_ref[...]
    o_ref[...] = jnp.reshape(jnp.transpose(z, (1, 0, 2)), (DIM, C_BLK))


_from_tiles = pl.pallas_call(
    _from_tiles_body,
    grid=(_TGRID,),
    in_specs=[pl.BlockSpec((TCG, DIM, 128), lambda j: (j, 0, 0))],
    out_specs=pl.BlockSpec((DIM, C_BLK), lambda j: (0, j)),
    out_shape=jax.ShapeDtypeStruct((DIM, N_NODES), jnp.float32),
)


_mesh = plsc.VectorSubcoreMesh(core_axis_name="c", subcore_axis_name="s")

_NEG = -2147483648


@functools.partial(
    pl.kernel,
    mesh=_mesh,
    compiler_params=pltpu.CompilerParams(
        needs_layout_passes=False, use_tc_tiling_on_sc=False),
    scratch_types=[
        pltpu.VMEM((BATCH,), jnp.int32),
        pltpu.VMEM((T_SIZE,), jnp.int32),
        pltpu.VMEM((BATCH,), jnp.int32),
        pltpu.VMEM((BATCH,), jnp.int32),
        pltpu.VMEM((64,), jnp.int32),
        pltpu.VMEM((64,), jnp.int32),
        pltpu.VMEM((64,), jnp.int32),
        pltpu.VMEM((SB,), jnp.int32),
        pltpu.VMEM((SB, DIM), jnp.float32),
        pltpu.VMEM((CELEMS,), jnp.float32),
        pltpu.SemaphoreType.DMA,
        pltpu.SemaphoreType.DMA,
    ],
)
def _sc_patch(nids_hbm, val_hbm, out_hbm,
              nids_v, t_v, wb_v, wn_v, hist_v, sts_v, ends_v,
              idxg, valrows, buf,
              sem_g, sem_c):
    wid = lax.axis_index("s") * NC + lax.axis_index("c")
    tc0 = jnp.minimum(wid * TCW, NTC)
    tc1 = jnp.minimum(tc0 + TCW, NTC)
    node0 = tc0 * 128
    node1 = jnp.minimum(tc1 * 128, N_NODES)
    iota = lax.iota(jnp.int32, L)
    neg1 = jnp.full((L,), -1, jnp.int32)
    zero16 = jnp.full((L,), 0, jnp.int32)

    pltpu.sync_copy(nids_hbm, nids_v)

    def init_body(i, carry):
        t_v[pl.ds(i * L, L)] = neg1
        return carry
    lax.fori_loop(0, T_SIZE // L, init_body, 0, unroll=4)
    for i in range(4):
        hist_v[pl.ds(i * L, L)] = zero16

    def stamp_body(i, carry):
        v = nids_v[pl.ds(i * L, L)]
        inr = (v >= node0) & (v < node1)
        _, last = plsc.scan_count(v, mask=inr)
        m = inr & last
        local = jnp.where(m, v - node0, 0)
        bidx = iota + i * L
        plsc.store_scatter(t_v, [local], bidx, mask=m)
        return carry
    lax.fori_loop(0, BATCH // L, stamp_body, 0, unroll=2)

    def compact_body(k, cnt):
        t = t_v[pl.ds(k * L, L)]
        m = t >= 0
        m_i32 = m.astype(jnp.int32)
        inc = plsc.cumsum(m_i32)
        pos = cnt + inc - m_i32
        nvec = node0 + k * L + iota
        plsc.store_scatter(wb_v, [pos], t, mask=m)
        plsc.store_scatter(wn_v, [pos], nvec, mask=m)
        return cnt + jnp.max(inc)
    cnt = lax.fori_loop(0, T_SIZE // L, compact_body, jnp.int32(0), unroll=2)

    @pl.when(cnt > 0)
    def _tail():
        ones = jnp.full((L,), 1, jnp.int32)

        def hist_body(k, carry):
            p = k * L + iota
            valid = p < cnt
            nv = wn_v[pl.ds(k * L, L)]
            cid = jnp.where(valid, (nv - node0) // CNODES, 63)
            plsc.addupdate_scatter(hist_v, [cid], ones, mask=valid)
            return carry
        lax.fori_loop(0, (cnt + L - 1) // L, hist_body, 0)

        carry0 = jnp.int32(0)
        for i in range(4):
            h = hist_v[pl.ds(i * L, L)]
            inc = plsc.cumsum(h)
            sts_v[pl.ds(i * L, L)] = carry0 + inc - h
            ends_v[pl.ds(i * L, L)] = carry0 + inc
            carry0 = carry0 + jnp.max(inc)

        def chunk_body(c, carry):
            al = (c // L) * L
            lane = c - al
            sva = sts_v[pl.ds(al, L)]
            eva = ends_v[pl.ds(al, L)]
            s_c = jnp.max(jnp.where(iota == lane, sva, _NEG))
            e_c = jnp.max(jnp.where(iota == lane, eva, _NEG))

            @pl.when(e_c > s_c)
            def _do_chunk():
                ccols = jnp.minimum(tc1 - (tc0 + c * CCOLS), CCOLS)
                hbase = (tc0 + c * CCOLS) * (DIM * 128)
                cn0 = node0 + c * CNODES

                @pl.when(ccols == CCOLS)
                def _in_full():
                    pltpu.async_copy(out_hbm.at[pl.ds(hbase, CELEMS)],
                                     buf, sem_c).wait()

                @pl.when(ccols < CCOLS)
                def _in_tail():
                    def dma_in(t, carry2):
                        pltpu.async_copy(
                            out_hbm.at[pl.ds(hbase + t * 2048, 2048)],
                            buf.at[pl.ds(t * 2048, 2048)], sem_c).wait()
                        return carry2
                    lax.fori_loop(0, ccols, dma_in, 0)

                def sb_body(g, carry3):
                    sb0 = s_c + g * SB
                    def gi_body(k, carry4):
                        p = sb0 + k * L + iota
                        pcl = jnp.minimum(p, e_c - 1)
                        idxg[pl.ds(k * L, L)] = plsc.load_gather(wb_v, [pcl])
                        return carry4
                    lax.fori_loop(0, SB // L, gi_body, 0)
                    pltpu.async_copy(val_hbm.at[idxg], valrows, sem_g).wait()

                    def pv_body(k, carry5):
                        p = sb0 + k * L + iota
                        pcl = jnp.minimum(p, e_c - 1)
                        nv = plsc.load_gather(wn_v, [pcl])
                        loc = nv - cn0
                        bases = (loc // 128) * (DIM * 128) + loc % 128
                        rows = pcl - sb0
                        for j in range(L):
                            sel = iota == j
                            bj = jnp.max(jnp.where(sel, bases, _NEG))
                            rj = jnp.max(jnp.where(sel, rows, _NEG))
                            rowv = plsc.load_gather(
                                valrows, [jnp.full((L,), 0, jnp.int32) + rj,
                                          iota])
                            plsc.store_scatter(
                                buf, [bj + iota * 128], rowv)
                        return carry5
                    nv_regs = (jnp.minimum(e_c - sb0, SB) + L - 1) // L
                    lax.fori_loop(0, nv_regs, pv_body, 0)
                    return carry3
                ngb = (e_c - s_c + SB - 1) // SB
                lax.fori_loop(0, ngb, sb_body, 0)

                @pl.when(ccols == CCOLS)
                def _out_full():
                    pltpu.async_copy(buf, out_hbm.at[pl.ds(hbase, CELEMS)],
                                     sem_c).wait()

                @pl.when(ccols < CCOLS)
                def _out_tail():
                    def dma_out(t, carry6):
                        pltpu.async_copy(
                            buf.at[pl.ds(t * 2048, 2048)],
                            out_hbm.at[pl.ds(hbase + t * 2048, 2048)],
                            sem_c).wait()
                        return carry6
                    lax.fori_loop(0, ccols, dma_out, 0)
            return carry
        lax.fori_loop(0, NCH, chunk_body, 0)


def kernel(memory, nids, val):
    yt = _to_tiles(memory.T)
    r = jax.new_ref(jnp.reshape(yt, (FLAT,)))
    _sc_patch(nids, val, r)
    z = jnp.reshape(r[...], (NTC, DIM, 128))
    return _from_tiles(z).T


# speedup vs baseline: 1.0737x; 1.0737x over previous
import functools

import jax
import jax.numpy as jnp
from jax import lax
from jax.experimental import pallas as pl
from jax.experimental.pallas import tpu as pltpu
from jax.experimental.pallas import tpu_sc as plsc

N_NODES = 1000000
DIM = 16
BATCH = 16384
L = 16

NC = 2
NS = 16
NW = NC * NS

NTC = (N_NODES + 127) // 128
FLAT = NTC * DIM * 128
TCW = (NTC + NW - 1) // NW
NODES_W = TCW * 128
T_SIZE = NODES_W

CCOLS = 12
CNODES = CCOLS * 128
CELEMS = CCOLS * DIM * 128
NCH = (TCW + CCOLS - 1) // CCOLS
SB = 128


TCG = 96
C_BLK = TCG * 128
_TGRID = (N_NODES + C_BLK - 1) // C_BLK


def _to_tiles_body(x_ref, o_ref):
    x = x_ref[...]
    o_ref[...] = jnp.transpose(jnp.reshape(x, (DIM, TCG, 128)), (1, 0, 2))


_to_tiles = pl.pallas_call(
    _to_tiles_body,
    grid=(_TGRID,),
    in_specs=[pl.BlockSpec((DIM, C_BLK), lambda j: (0, j))],
    out_specs=pl.BlockSpec((TCG, DIM, 128), lambda j: (j, 0, 0)),
    out_shape=jax.ShapeDtypeStruct((NTC, DIM, 128), jnp.float32),
)


def _from_tiles_body(z_ref, o_ref):
    z = z_ref[...]
    o_ref[...] = jnp.reshape(jnp.transpose(z, (1, 0, 2)), (DIM, C_BLK))


_from_tiles = pl.pallas_call(
    _from_tiles_body,
    grid=(_TGRID,),
    in_specs=[pl.BlockSpec((TCG, DIM, 128), lambda j: (j, 0, 0))],
    out_specs=pl.BlockSpec((DIM, C_BLK), lambda j: (0, j)),
    out_shape=jax.ShapeDtypeStruct((DIM, N_NODES), jnp.float32),
)


_mesh = plsc.VectorSubcoreMesh(core_axis_name="c", subcore_axis_name="s")

_NEG = -2147483648


@functools.partial(
    pl.kernel,
    mesh=_mesh,
    compiler_params=pltpu.CompilerParams(
        needs_layout_passes=False, use_tc_tiling_on_sc=False),
    scratch_types=[
        pltpu.VMEM((BATCH,), jnp.int32),
        pltpu.VMEM((T_SIZE,), jnp.int32),
        pltpu.VMEM((BATCH,), jnp.int32),
        pltpu.VMEM((BATCH,), jnp.int32),
        pltpu.VMEM((64,), jnp.int32),
        pltpu.VMEM((64,), jnp.int32),
        pltpu.VMEM((64,), jnp.int32),
        pltpu.VMEM((SB,), jnp.int32),
        pltpu.VMEM((SB, DIM), jnp.float32),
        pltpu.VMEM((CELEMS,), jnp.float32),
        pltpu.SemaphoreType.DMA,
        pltpu.SemaphoreType.DMA,
    ],
)
def _sc_patch(nids_hbm, val_hbm, out_hbm,
              nids_v, t_v, wb_v, wn_v, hist_v, sts_v, ends_v,
              idxg, valrows, buf,
              sem_g, sem_c):
    wid = lax.axis_index("s") * NC + lax.axis_index("c")
    tc0 = jnp.minimum(wid * TCW, NTC)
    tc1 = jnp.minimum(tc0 + TCW, NTC)
    node0 = tc0 * 128
    node1 = jnp.minimum(tc1 * 128, N_NODES)
    iota = lax.iota(jnp.int32, L)
    neg1 = jnp.full((L,), -1, jnp.int32)
    zero16 = jnp.full((L,), 0, jnp.int32)

    pltpu.sync_copy(nids_hbm, nids_v)

    def init_body(i, carry):
        t_v[pl.ds(i * L, L)] = neg1
        return carry
    lax.fori_loop(0, T_SIZE // L, init_body, 0, unroll=4)
    for i in range(4):
        hist_v[pl.ds(i * L, L)] = zero16

    def stamp_body(i, carry):
        v = nids_v[pl.ds(i * L, L)]
        inr = (v >= node0) & (v < node1)
        _, last = plsc.scan_count(v, mask=inr)
        m = inr & last
        local = jnp.where(m, v - node0, 0)
        bidx = iota + i * L
        plsc.store_scatter(t_v, [local], bidx, mask=m)
        return carry
    lax.fori_loop(0, BATCH // L, stamp_body, 0, unroll=2)

    def compact_body(k, cnt):
        t = t_v[pl.ds(k * L, L)]
        m = t >= 0
        m_i32 = m.astype(jnp.int32)
        inc = plsc.cumsum(m_i32)
        pos = cnt + inc - m_i32
        nvec = node0 + k * L + iota
        plsc.store_scatter(wb_v, [pos], t, mask=m)
        plsc.store_scatter(wn_v, [pos], nvec, mask=m)
        return cnt + jnp.max(inc)
    cnt = lax.fori_loop(0, T_SIZE // L, compact_body, jnp.int32(0), unroll=2)

    @pl.when(cnt > 0)
    def _tail():
        ones = jnp.full((L,), 1, jnp.int32)

        def hist_body(k, carry):
            p = k * L + iota
            valid = p < cnt
            nv = wn_v[pl.ds(k * L, L)]
            cid = jnp.where(valid, (nv - node0) // CNODES, 63)
            plsc.addupdate_scatter(hist_v, [cid], ones, mask=valid)
            return carry
        lax.fori_loop(0, (cnt + L - 1) // L, hist_body, 0)

        carry0 = jnp.int32(0)
        for i in range(4):
            h = hist_v[pl.ds(i * L, L)]
            inc = plsc.cumsum(h)
            sts_v[pl.ds(i * L, L)] = carry0 + inc - h
            ends_v[pl.ds(i * L, L)] = carry0 + inc
            carry0 = carry0 + jnp.max(inc)

        def chunk_body(c, carry):
            al = (c // L) * L
            lane = c - al
            sva = sts_v[pl.ds(al, L)]
            eva = ends_v[pl.ds(al, L)]
            s_c = jnp.max(jnp.where(iota == lane, sva, _NEG))
            e_c = jnp.max(jnp.where(iota == lane, eva, _NEG))

            @pl.when(e_c > s_c)
            def _do_chunk():
                ccols = jnp.minimum(tc1 - (tc0 + c * CCOLS), CCOLS)
                hbase = (tc0 + c * CCOLS) * (DIM * 128)
                cn0 = node0 + c * CNODES

                @pl.when(ccols == CCOLS)
                def _in_full():
                    pltpu.async_copy(out_hbm.at[pl.ds(hbase, CELEMS)],
                                     buf, sem_c).wait()

                @pl.when(ccols < CCOLS)
                def _in_tail():
                    def dma_in(t, carry2):
                        pltpu.async_copy(
                            out_hbm.at[pl.ds(hbase + t * 2048, 2048)],
                            buf.at[pl.ds(t * 2048, 2048)], sem_c).wait()
                        return carry2
                    lax.fori_loop(0, ccols, dma_in, 0)

                def sb_body(g, carry3):
                    sb0 = s_c + g * SB
                    def gi_body(k, carry4):
                        p = sb0 + k * L + iota
                        pcl = jnp.minimum(p, e_c - 1)
                        idxg[pl.ds(k * L, L)] = plsc.load_gather(wb_v, [pcl])
                        return carry4
                    lax.fori_loop(0, SB // L, gi_body, 0)
                    pltpu.async_copy(val_hbm.at[idxg], valrows, sem_g).wait()

                    def pv_body(k, carry5):
                        p = sb0 + k * L + iota
                        pcl = jnp.minimum(p, e_c - 1)
                        nv = plsc.load_gather(wn_v, [pcl])
                        loc = nv - cn0
                        bases = (loc // 128) * (DIM * 128) + loc % 128
                        rows = pcl - sb0
                        for j in range(L):
                            sel = iota == j
                            bj = jnp.max(jnp.where(sel, bases, _NEG))
                            rj = jnp.max(jnp.where(sel, rows, _NEG))
                            rowv = plsc.load_gather(
                                valrows, [jnp.full((L,), 0, jnp.int32) + rj,
                                          iota])
                            plsc.store_scatter(
                                buf, [bj + iota * 128], rowv)
                        return carry5
                    nv_regs = (jnp.minimum(e_c - sb0, SB) + L - 1) // L
                    lax.fori_loop(0, nv_regs, pv_body, 0)
                    return carry3
                ngb = (e_c - s_c + SB - 1) // SB
                lax.fori_loop(0, ngb, sb_body, 0)

                @pl.when(ccols == CCOLS)
                def _out_full():
                    pltpu.async_copy(buf, out_hbm.at[pl.ds(hbase, CELEMS)],
                                     sem_c).wait()

                @pl.when(ccols < CCOLS)
                def _out_tail():
                    def dma_out(t, carry6):
                        pltpu.async_copy(
                            buf.at[pl.ds(t * 2048, 2048)],
                            out_hbm.at[pl.ds(hbase + t * 2048, 2048)],
                            sem_c).wait()
                        return carry6
                    lax.fori_loop(0, ccols, dma_out, 0)
            return carry
        lax.fori_loop(0, NCH, chunk_body, 0)


def kernel(memory, nids, val):
    yt = _to_tiles(memory.T)
    r = jax.new_ref(jnp.reshape(yt, (FLAT,)))
    _sc_patch(nids, val, r)
    z = jnp.reshape(r[...], (NTC, DIM, 128))
    return _from_tiles(z).T
